# Initial kernel scaffold; baseline (speedup 1.0000x reference)
import jax
import jax.numpy as jnp
from jax.experimental import pallas as pl


def kernel(input_ids, table1, table2, table3, W):
    raise NotImplementedError("write your pallas kernel here")


# speedup vs baseline: 8.1191x; 8.1191x over previous
import functools

import jax
import jax.numpy as jnp
from jax import lax
from jax.experimental import pallas as pl
from jax.experimental.pallas import tpu as pltpu
from jax.experimental.pallas import tpu_sc as plsc

_TABLE = 1000000
_D = 32
_H = 128
_NC, _NS = 2, 16
_NW = _NC * _NS
_CH = 128

_C2_HI, _C2_LO = 242496, 104729
_C3_HI, _C3_LO = 935232, 97593


def _sc_gather(ids, prev, t1, t2, t3, n):
    npw = n // _NW
    nch = npw // _CH
    mesh = plsc.VectorSubcoreMesh(core_axis_name="c", subcore_axis_name="s")

    @functools.partial(
        pl.kernel,
        out_type=[jax.ShapeDtypeStruct((n, _D), jnp.float32)] * 3,
        mesh=mesh,
        scratch_types=[
            pltpu.VMEM((_CH,), jnp.int32),
            pltpu.VMEM((_CH,), jnp.int32),
            pltpu.VMEM((_CH,), jnp.int32),
            pltpu.VMEM((_CH,), jnp.int32),
            pltpu.VMEM((_CH,), jnp.int32),
            pltpu.VMEM((_CH, _D), jnp.float32),
            pltpu.VMEM((_CH, _D), jnp.float32),
            pltpu.VMEM((_CH, _D), jnp.float32),
            pltpu.SemaphoreType.DMA,
        ],
        compiler_params=pltpu.CompilerParams(use_tc_tiling_on_sc=False),
    )
    def k(ids_h, prev_h, t1_h, t2_h, t3_h, e1_h, e2_h, e3_h,
          ids_v, prev_v, i1_v, i2_v, i3_v, r1_v, r2_v, r3_v, sem):
        wid = lax.axis_index("s") * _NC + lax.axis_index("c")
        wbase = wid * jnp.int32(npw)

        def chunk(c, carry):
            base = wbase + c * jnp.int32(_CH)
            pltpu.sync_copy(ids_h.at[pl.ds(base, _CH)], ids_v)
            pltpu.sync_copy(prev_h.at[pl.ds(base, _CH)], prev_v)
            for i in range(_CH // 16):
                sl = pl.ds(i * 16, 16)
                cur = ids_v[sl]
                prv = prev_v[sl]
                p_hi = lax.shift_right_logical(prv, jnp.int32(10))
                p_lo = lax.bitwise_and(prv, jnp.int32(1023))
                i1_v[sl] = (prv * 8191 + cur) % _TABLE
                i2_v[sl] = (p_hi * _C2_HI + p_lo * _C2_LO + cur) % _TABLE
                i3_v[sl] = (p_hi * _C3_HI + p_lo * _C3_LO + cur) % _TABLE
            c1 = pltpu.async_copy(t1_h.at[i1_v], r1_v, sem)
            c2 = pltpu.async_copy(t2_h.at[i2_v], r2_v, sem)
            c3 = pltpu.async_copy(t3_h.at[i3_v], r3_v, sem)
            c1.wait()
            c2.wait()
            c3.wait()
            pltpu.sync_copy(r1_v, e1_h.at[pl.ds(base, _CH)])
            pltpu.sync_copy(r2_v, e2_h.at[pl.ds(base, _CH)])
            pltpu.sync_copy(r3_v, e3_h.at[pl.ds(base, _CH)])
            return carry

        lax.fori_loop(jnp.int32(0), jnp.int32(nch), chunk, jnp.int32(0))

    return k(ids, prev, t1, t2, t3)


def _tc_project(e1, e2, e3, w, n):
    bm = 2048

    def body(e1_r, e2_r, e3_r, w_r, o_r):
        cat = jnp.concatenate([e1_r[...], e2_r[...], e3_r[...]], axis=1)
        o_r[...] = lax.dot_general(
            cat, w_r[...], (((1,), (1,)), ((), ())),
            preferred_element_type=jnp.float32)

    return pl.pallas_call(
        body,
        grid=(n // bm,),
        in_specs=[
            pl.BlockSpec((bm, _D), lambda i: (i, jnp.int32(0))),
            pl.BlockSpec((bm, _D), lambda i: (i, jnp.int32(0))),
            pl.BlockSpec((bm, _D), lambda i: (i, jnp.int32(0))),
            pl.BlockSpec((_H, 3 * _D), lambda i: (jnp.int32(0), jnp.int32(0))),
        ],
        out_specs=pl.BlockSpec((bm, _H), lambda i: (i, jnp.int32(0))),
        out_shape=jax.ShapeDtypeStruct((n, _H), jnp.float32),
    )(e1, e2, e3, w)


def kernel(input_ids, table1, table2, table3, W):
    b, t = input_ids.shape
    n = b * t
    ids32 = input_ids.astype(jnp.int32)
    prev = jnp.concatenate(
        [jnp.zeros((b, 1), jnp.int32), ids32[:, :-1]], axis=1)
    e1, e2, e3 = _sc_gather(
        ids32.reshape(-1), prev.reshape(-1), table1, table2, table3, n)
    out = _tc_project(e1, e2, e3, W, n)
    return out.reshape(b, t, _H)
